# baseline (speedup 1.0000x reference)
import functools

import jax
import jax.numpy as jnp
from jax import lax
from jax.experimental import pallas as pl
from jax.experimental.pallas import tpu as pltpu
from jax.experimental.pallas import tpu_sc as plsc

NN = 10000
RR = 8
DD = 128
EE = 320000
NB = 2
NC = 2
NS = 16
LL = 16
NW = NC * NS
SEG = NN * RR
EPT = EE // NW
CH = 80
NCHUNK = EPT // CH
GRP = CH // LL
RPT = NN // NS

_MESH = plsc.VectorSubcoreMesh(
    core_axis_name="c", subcore_axis_name="s", num_cores=NC, num_subcores=NS)
_SC_PARAMS = pltpu.CompilerParams(needs_layout_passes=False)


def _count_body(dst_hbm, et_hbm, out_hbm, dstb, etb, hist):
  cid = lax.axis_index("c")
  sid = lax.axis_index("s")
  wid = sid * NC + cid
  zeros = jnp.zeros((LL,), jnp.int32)
  ones = jnp.ones((LL,), jnp.int32)
  for b in range(NB):
    def zbody(i, _):
      hist[pl.ds(i * LL, LL)] = zeros
      return 0
    lax.fori_loop(0, SEG // LL, zbody, 0)
    base = b * EE + wid * EPT
    pltpu.sync_copy(dst_hbm.at[pl.ds(base, EPT)], dstb)
    pltpu.sync_copy(et_hbm.at[pl.ds(base, EPT)], etb)
    def cbody(i, _):
      sl = pl.ds(i * LL, LL)
      seg = dstb[sl] * RR + etb[sl]
      plsc.addupdate_scatter(hist, [seg], ones)
      return 0
    lax.fori_loop(0, EPT // LL, cbody, 0)
    pltpu.sync_copy(hist, out_hbm.at[pl.ds((b * NW + wid) * SEG, SEG)])


_count = pl.kernel(
    _count_body,
    out_type=jax.ShapeDtypeStruct((NB * NW * SEG,), jnp.int32),
    mesh=_MESH,
    scratch_types=[
        pltpu.VMEM((EPT,), jnp.int32),
        pltpu.VMEM((EPT,), jnp.int32),
        pltpu.VMEM((SEG,), jnp.int32),
    ],
    compiler_params=_SC_PARAMS,
)


PAD_E = 327680
EPT2 = PAD_E // NW
CH2 = 128
SUP2 = 2048
NSUP2 = EPT2 // SUP2
SUPCH2 = SUP2 // CH2
GRP2 = CH2 // LL
NBUF = 2


def _expand_body(src_hbm, et_hbm, dst_hbm, inv_hbm, g_hbm, inve_hbm,
                 srcb, etb, dstb, invt, outg, outv):
  cid = lax.axis_index("c")
  sid = lax.axis_index("s")
  wid = sid * NC + cid
  lanes = lax.iota(jnp.int32, LL)
  for b in range(NB):
    pltpu.sync_copy(inv_hbm.at[pl.ds(b * SEG, SEG)], invt)
    base = wid * EPT2
    def sup_body(s, _):
      sbase = base + s * SUP2
      pltpu.sync_copy(src_hbm.at[pl.ds(b * PAD_E + sbase, SUP2)], srcb)
      pltpu.sync_copy(et_hbm.at[pl.ds(b * PAD_E + sbase, SUP2)], etb)
      pltpu.sync_copy(dst_hbm.at[pl.ds(b * PAD_E + sbase, SUP2)], dstb)
      def ebody(i, _):
        sl = pl.ds(i * LL, LL)
        t = etb[sl]
        seg = dstb[sl] * RR + t
        iv = plsc.load_gather(invt, [seg])
        eidx = (sbase + i * LL) + lanes
        outv[sl] = jnp.where(eidx < EE, iv, 0.0)
        outg[sl] = srcb[sl] * RR + t
        return 0
      lax.fori_loop(0, SUP2 // LL, ebody, 0)
      pltpu.sync_copy(outg, g_hbm.at[pl.ds(b * PAD_E + sbase, SUP2)])
      pltpu.sync_copy(outv, inve_hbm.at[pl.ds(b * PAD_E + sbase, SUP2)])
      return 0
    lax.fori_loop(0, NSUP2, sup_body, 0)


_expand = pl.kernel(
    _expand_body,
    out_type=(jax.ShapeDtypeStruct((NB * PAD_E,), jnp.int32),
              jax.ShapeDtypeStruct((NB * PAD_E,), jnp.float32)),
    mesh=_MESH,
    scratch_types=[
        pltpu.VMEM((SUP2,), jnp.int32),
        pltpu.VMEM((SUP2,), jnp.int32),
        pltpu.VMEM((SUP2,), jnp.int32),
        pltpu.VMEM((SEG,), jnp.float32),
        pltpu.VMEM((SUP2,), jnp.int32),
        pltpu.VMEM((SUP2,), jnp.float32),
    ],
    compiler_params=_SC_PARAMS,
)


def _scat_body(y_hbm, g_hbm, inve_hbm, dst5_hbm, z_hbm,
               out_hbm, gbs, invsb, dstb, rows, acc, gsem, ssem):
  cid = lax.axis_index("c")
  sid = lax.axis_index("s")
  for b in range(NB):
    pltpu.sync_copy(z_hbm, acc.at[pl.ds(sid * RPT, RPT)])
    plsc.subcore_barrier()
    yb = y_hbm.at[b]
    base = b * PAD_E + cid * (PAD_E // NC) + sid * EPT2
    def sup_body(s, _):
      sbase = base + s * SUP2
      pltpu.sync_copy(g_hbm.at[pl.ds(sbase, SUP2)], gbs)
      pltpu.sync_copy(inve_hbm.at[pl.ds(sbase, SUP2)], invsb)
      pltpu.sync_copy(dst5_hbm.at[b, cid, sid, pl.ds(s * SUPCH2, SUPCH2)],
                      dstb)
      def cbody(i, _):
        pltpu.async_copy(yb.at[gbs.at[pl.ds(i * CH2, CH2)]], rows,
                         gsem).wait()
        def sc_body(k, _):
          for j in range(LL):
            e = k * LL + j
            scal = plsc.load_gather(
                invsb, [jnp.full((LL,), i * CH2 + e, jnp.int32)])
            for q in range(DD // LL):
              sl = pl.ds(q * LL, LL)
              rows[e, sl] = rows[e, sl] * scal
          return 0
        lax.fori_loop(0, GRP2, sc_body, 0)
        pltpu.sync_copy(rows, acc.at[dstb.at[i]], add=True)
        return 0
      lax.fori_loop(0, SUPCH2, cbody, 0)
      return 0
    lax.fori_loop(0, NSUP2, sup_body, 0)
    plsc.subcore_barrier()
    pltpu.sync_copy(acc.at[pl.ds(sid * RPT, RPT)], out_hbm.at[b, cid, sid])


_scat = pl.kernel(
    _scat_body,
    out_type=jax.ShapeDtypeStruct((NB, NC, NS, RPT, DD), jnp.float32),
    mesh=_MESH,
    scratch_types=[
        pltpu.VMEM((SUP2,), jnp.int32),
        pltpu.VMEM((SUP2,), jnp.float32),
        pltpu.VMEM((SUPCH2, CH2), jnp.int32),
        pltpu.VMEM((CH2, DD), jnp.float32),
        pltpu.VMEM_SHARED((NN, DD), jnp.float32),
        pltpu.SemaphoreType.DMA,
        pltpu.SemaphoreType.DMA,
    ],
    compiler_params=_SC_PARAMS,
)


def _inv_body(cnt_ref, inv_ref):
  s = jnp.sum(cnt_ref[0], axis=0)
  inv_ref[...] = (1.0 / jnp.maximum(s, 1).astype(jnp.float32))[None]


def _inv(cnt):
  cnt = cnt.reshape(NB, NW, SEG // DD, DD)
  out = pl.pallas_call(
      _inv_body,
      out_shape=jax.ShapeDtypeStruct((NB, SEG // DD, DD), jnp.float32),
      grid=(NB,),
      in_specs=[pl.BlockSpec((1, NW, SEG // DD, DD), lambda b: (b, 0, 0, 0))],
      out_specs=pl.BlockSpec((1, SEG // DD, DD), lambda b: (b, 0, 0)),
  )(cnt)
  return out.reshape(NB, SEG)


_BM = 2000


def _mm_body(x_ref, w_ref, r_ref, bias_ref, y_ref, xr_ref):
  xb = x_ref[0]
  y_ref[...] = jnp.dot(xb, w_ref[...], preferred_element_type=jnp.float32)[None]
  xr_ref[...] = (jnp.dot(xb, r_ref[...], preferred_element_type=jnp.float32)
                 + bias_ref[...])[None]


def _mm(x, wc, root, bias):
  return pl.pallas_call(
      _mm_body,
      out_shape=(
          jax.ShapeDtypeStruct((NB, NN, RR * DD), jnp.float32),
          jax.ShapeDtypeStruct((NB, NN, DD), jnp.float32),
      ),
      grid=(NB, NN // _BM),
      in_specs=[
          pl.BlockSpec((1, _BM, DD), lambda b, i: (b, i, 0)),
          pl.BlockSpec((DD, RR * DD), lambda b, i: (0, 0)),
          pl.BlockSpec((DD, DD), lambda b, i: (0, 0)),
          pl.BlockSpec((1, DD), lambda b, i: (0, 0)),
      ],
      out_specs=(
          pl.BlockSpec((1, _BM, RR * DD), lambda b, i: (b, i, 0)),
          pl.BlockSpec((1, _BM, DD), lambda b, i: (b, i, 0)),
      ),
  )(x, wc, root, bias)


def _fuse_body(part_ref, xr_ref, o_ref, *, relu):
  s = part_ref[0, 0] + part_ref[0, 1] + xr_ref[0]
  if relu:
    s = jnp.maximum(s, 0.0)
  o_ref[...] = s[None]


def _fuse(parts, xr, relu):
  return pl.pallas_call(
      functools.partial(_fuse_body, relu=relu),
      out_shape=jax.ShapeDtypeStruct((NB, NN, DD), jnp.float32),
      grid=(NB, NN // _BM),
      in_specs=[
          pl.BlockSpec((1, NC, _BM, DD), lambda b, i: (b, 0, i, 0)),
          pl.BlockSpec((1, _BM, DD), lambda b, i: (b, i, 0)),
      ],
      out_specs=pl.BlockSpec((1, _BM, DD), lambda b, i: (b, i, 0)),
  )(parts, xr)


def kernel(x, batched_edge_index, batched_edge_type, W1, root1, b1,
           W2, root2, b2):
  src2 = batched_edge_index[:, 0, :].astype(jnp.int32)
  dst2 = batched_edge_index[:, 1, :].astype(jnp.int32)
  et2 = batched_edge_type.astype(jnp.int32)
  pad = ((0, 0), (0, PAD_E - EE))
  src_p = jnp.pad(src2, pad).reshape(NB * PAD_E)
  dst_p = jnp.pad(dst2, pad).reshape(NB * PAD_E)
  et_p = jnp.pad(et2, pad).reshape(NB * PAD_E)
  dst5 = dst_p.reshape(NB, NC, NS, EPT2 // CH2, CH2)
  zrows = jnp.zeros((RPT, DD), jnp.float32)

  cnt = _count(dst2.reshape(NB * EE), et2.reshape(NB * EE))
  inv = _inv(cnt).reshape(NB * SEG)
  g_e, inv_e = _expand(src_p, et_p, dst_p, inv)

  wc1 = W1.transpose(1, 0, 2).reshape(DD, RR * DD)
  wc2 = W2.transpose(1, 0, 2).reshape(DD, RR * DD)

  y1, xr1 = _mm(x, wc1, root1, b1.reshape(1, DD))
  parts1 = _scat(y1.reshape(NB, SEG, DD), g_e, inv_e, dst5, zrows)
  h = _fuse(parts1.reshape(NB, NC, NN, DD), xr1, relu=True)

  y2, xr2 = _mm(h, wc2, root2, b2.reshape(1, DD))
  parts2 = _scat(y2.reshape(NB, SEG, DD), g_e, inv_e, dst5, zrows)
  return _fuse(parts2.reshape(NB, NC, NN, DD), xr2, relu=False)


# speedup vs baseline: 1.8872x; 1.8872x over previous
import functools

import jax
import jax.numpy as jnp
from jax import lax
from jax.experimental import pallas as pl
from jax.experimental.pallas import tpu as pltpu
from jax.experimental.pallas import tpu_sc as plsc

NN = 10000
RR = 8
DD = 128
EE = 320000
NB = 2
NC = 2
NS = 16
LL = 16
NW = NC * NS
SEG = NN * RR
EPT = EE // NW
CH = 80
NCHUNK = EPT // CH
GRP = CH // LL
RPT = NN // NS

_MESH = plsc.VectorSubcoreMesh(
    core_axis_name="c", subcore_axis_name="s", num_cores=NC, num_subcores=NS)
_SC_PARAMS = pltpu.CompilerParams(needs_layout_passes=False)


def _count_body(dst_hbm, et_hbm, out_hbm, dstb, etb, hist):
  cid = lax.axis_index("c")
  sid = lax.axis_index("s")
  wid = sid * NC + cid
  zeros = jnp.zeros((LL,), jnp.int32)
  ones = jnp.ones((LL,), jnp.int32)
  for b in range(NB):
    def zbody(i, _):
      hist[pl.ds(i * LL, LL)] = zeros
      return 0
    lax.fori_loop(0, SEG // LL, zbody, 0)
    base = b * EE + wid * EPT
    pltpu.sync_copy(dst_hbm.at[pl.ds(base, EPT)], dstb)
    pltpu.sync_copy(et_hbm.at[pl.ds(base, EPT)], etb)
    def cbody(i, _):
      sl = pl.ds(i * LL, LL)
      seg = dstb[sl] * RR + etb[sl]
      plsc.addupdate_scatter(hist, [seg], ones)
      return 0
    lax.fori_loop(0, EPT // LL, cbody, 0)
    pltpu.sync_copy(hist, out_hbm.at[pl.ds((b * NW + wid) * SEG, SEG)])


_count = pl.kernel(
    _count_body,
    out_type=jax.ShapeDtypeStruct((NB * NW * SEG,), jnp.int32),
    mesh=_MESH,
    scratch_types=[
        pltpu.VMEM((EPT,), jnp.int32),
        pltpu.VMEM((EPT,), jnp.int32),
        pltpu.VMEM((SEG,), jnp.int32),
    ],
    compiler_params=_SC_PARAMS,
)


PAD_E = 327680
EPT2 = PAD_E // NW
CH2 = 128
SUP2 = 2048
NSUP2 = EPT2 // SUP2
SUPCH2 = SUP2 // CH2
GRP2 = CH2 // LL
NBUF = 2


def _expand_body(src_hbm, et_hbm, dst_hbm, inv_hbm, g_hbm, inve_hbm,
                 srcb, etb, dstb, invt, outg, outv):
  cid = lax.axis_index("c")
  sid = lax.axis_index("s")
  wid = sid * NC + cid
  lanes = lax.iota(jnp.int32, LL)
  for b in range(NB):
    pltpu.sync_copy(inv_hbm.at[pl.ds(b * SEG, SEG)], invt)
    base = wid * EPT2
    def sup_body(s, _):
      sbase = base + s * SUP2
      pltpu.sync_copy(src_hbm.at[pl.ds(b * PAD_E + sbase, SUP2)], srcb)
      pltpu.sync_copy(et_hbm.at[pl.ds(b * PAD_E + sbase, SUP2)], etb)
      pltpu.sync_copy(dst_hbm.at[pl.ds(b * PAD_E + sbase, SUP2)], dstb)
      def ebody(i, _):
        sl = pl.ds(i * LL, LL)
        t = etb[sl]
        seg = dstb[sl] * RR + t
        iv = plsc.load_gather(invt, [seg])
        eidx = (sbase + i * LL) + lanes
        outv[sl] = jnp.where(eidx < EE, iv, 0.0)
        outg[sl] = srcb[sl] * RR + t
        return 0
      lax.fori_loop(0, SUP2 // LL, ebody, 0)
      pltpu.sync_copy(outg, g_hbm.at[pl.ds(b * PAD_E + sbase, SUP2)])
      pltpu.sync_copy(outv, inve_hbm.at[pl.ds(b * PAD_E + sbase, SUP2)])
      return 0
    lax.fori_loop(0, NSUP2, sup_body, 0)


_expand = pl.kernel(
    _expand_body,
    out_type=(jax.ShapeDtypeStruct((NB * PAD_E,), jnp.int32),
              jax.ShapeDtypeStruct((NB * PAD_E,), jnp.float32)),
    mesh=_MESH,
    scratch_types=[
        pltpu.VMEM((SUP2,), jnp.int32),
        pltpu.VMEM((SUP2,), jnp.int32),
        pltpu.VMEM((SUP2,), jnp.int32),
        pltpu.VMEM((SEG,), jnp.float32),
        pltpu.VMEM((SUP2,), jnp.int32),
        pltpu.VMEM((SUP2,), jnp.float32),
    ],
    compiler_params=_SC_PARAMS,
)


def _scat_body(y_hbm, g_hbm, inve_hbm, dst5_hbm, z_hbm,
               out_hbm, gbs, invsb, dstb, rows, acc, gsem, ssem):
  cid = lax.axis_index("c")
  sid = lax.axis_index("s")
  for b in range(NB):
    pltpu.sync_copy(z_hbm, acc.at[pl.ds(sid * RPT, RPT)])
    plsc.subcore_barrier()
    yb = y_hbm.at[b]
    base = b * PAD_E + cid * (PAD_E // NC) + sid * EPT2
    def sup_body(s, _):
      sbase = base + s * SUP2
      pltpu.sync_copy(g_hbm.at[pl.ds(sbase, SUP2)], gbs)
      pltpu.sync_copy(inve_hbm.at[pl.ds(sbase, SUP2)], invsb)
      pltpu.sync_copy(dst5_hbm.at[b, cid, sid, pl.ds(s * SUPCH2, SUPCH2)],
                      dstb)
      def cbody(i, _):
        pltpu.async_copy(yb.at[gbs.at[pl.ds(i * CH2, CH2)]], rows,
                         gsem).wait()
        def sc_body(k, _):
          for j in range(LL):
            e = k * LL + j
            scal = plsc.load_gather(
                invsb, [jnp.full((LL,), i * CH2 + e, jnp.int32)])
            for q in range(DD // LL):
              sl = pl.ds(q * LL, LL)
              rows[e, sl] = rows[e, sl] * scal
          return 0
        lax.fori_loop(0, GRP2, sc_body, 0)
        pltpu.sync_copy(rows, acc.at[dstb.at[i]], add=True)
        return 0
      lax.fori_loop(0, SUPCH2, cbody, 0)
      return 0
    lax.fori_loop(0, NSUP2, sup_body, 0)
    plsc.subcore_barrier()
    pltpu.sync_copy(acc.at[pl.ds(sid * RPT, RPT)], out_hbm.at[b, cid, sid])


_scat = pl.kernel(
    _scat_body,
    out_type=jax.ShapeDtypeStruct((NB, NC, NS, RPT, DD), jnp.float32),
    mesh=_MESH,
    scratch_types=[
        pltpu.VMEM((SUP2,), jnp.int32),
        pltpu.VMEM((SUP2,), jnp.float32),
        pltpu.VMEM((SUPCH2, CH2), jnp.int32),
        pltpu.VMEM((CH2, DD), jnp.float32),
        pltpu.VMEM_SHARED((NN, DD), jnp.float32),
        pltpu.SemaphoreType.DMA,
        pltpu.SemaphoreType.DMA,
    ],
    compiler_params=_SC_PARAMS,
)


def _inv_body(cnt_ref, inv_ref):
  s = jnp.sum(cnt_ref[0], axis=0)
  inv_ref[...] = (1.0 / jnp.maximum(s, 1).astype(jnp.float32))[None]


def _inv(cnt):
  cnt = cnt.reshape(NB, NW, SEG // DD, DD)
  out = pl.pallas_call(
      _inv_body,
      out_shape=jax.ShapeDtypeStruct((NB, SEG // DD, DD), jnp.float32),
      grid=(NB,),
      in_specs=[pl.BlockSpec((1, NW, SEG // DD, DD), lambda b: (b, 0, 0, 0))],
      out_specs=pl.BlockSpec((1, SEG // DD, DD), lambda b: (b, 0, 0)),
  )(cnt)
  return out.reshape(NB, SEG)


_BM = 2000


def _mm_body(x_ref, w_ref, r_ref, bias_ref, y_ref, xr_ref):
  xb = x_ref[0]
  y_ref[...] = jnp.dot(xb, w_ref[...], preferred_element_type=jnp.float32)[None]
  xr_ref[...] = (jnp.dot(xb, r_ref[...], preferred_element_type=jnp.float32)
                 + bias_ref[...])[None]


def _mm(x, wc, root, bias):
  return pl.pallas_call(
      _mm_body,
      out_shape=(
          jax.ShapeDtypeStruct((NB, NN, RR * DD), jnp.float32),
          jax.ShapeDtypeStruct((NB, NN, DD), jnp.float32),
      ),
      grid=(NB, NN // _BM),
      in_specs=[
          pl.BlockSpec((1, _BM, DD), lambda b, i: (b, i, 0)),
          pl.BlockSpec((DD, RR * DD), lambda b, i: (0, 0)),
          pl.BlockSpec((DD, DD), lambda b, i: (0, 0)),
          pl.BlockSpec((1, DD), lambda b, i: (0, 0)),
      ],
      out_specs=(
          pl.BlockSpec((1, _BM, RR * DD), lambda b, i: (b, i, 0)),
          pl.BlockSpec((1, _BM, DD), lambda b, i: (b, i, 0)),
      ),
  )(x, wc, root, bias)


def _fuse_body(part_ref, xr_ref, o_ref, *, relu):
  s = part_ref[0, 0] + part_ref[0, 1] + xr_ref[0]
  if relu:
    s = jnp.maximum(s, 0.0)
  o_ref[...] = s[None]


def _fuse(parts, xr, relu):
  return pl.pallas_call(
      functools.partial(_fuse_body, relu=relu),
      out_shape=jax.ShapeDtypeStruct((NB, NN, DD), jnp.float32),
      grid=(NB, NN // _BM),
      in_specs=[
          pl.BlockSpec((1, NC, _BM, DD), lambda b, i: (b, 0, i, 0)),
          pl.BlockSpec((1, _BM, DD), lambda b, i: (b, i, 0)),
      ],
      out_specs=pl.BlockSpec((1, _BM, DD), lambda b, i: (b, i, 0)),
  )(parts, xr)


def kernel(x, batched_edge_index, batched_edge_type, W1, root1, b1,
           W2, root2, b2):
  src2 = batched_edge_index[:, 0, :].astype(jnp.int32)
  dst2 = batched_edge_index[:, 1, :].astype(jnp.int32)
  et2 = batched_edge_type.astype(jnp.int32)
  spread = (jnp.arange(PAD_E - EE, dtype=jnp.int32) % NN)[None, :]
  spread = jnp.broadcast_to(spread, (NB, PAD_E - EE))
  src_p = jnp.concatenate([src2, spread], axis=1).reshape(NB * PAD_E)
  dst_p = jnp.concatenate([dst2, spread], axis=1).reshape(NB * PAD_E)
  et_p = jnp.concatenate(
      [et2, jnp.zeros((NB, PAD_E - EE), jnp.int32)], axis=1
  ).reshape(NB * PAD_E)
  dst5 = dst_p.reshape(NB, NC, NS, EPT2 // CH2, CH2)
  zrows = jnp.zeros((RPT, DD), jnp.float32)

  cnt = _count(dst2.reshape(NB * EE), et2.reshape(NB * EE))
  inv = _inv(cnt).reshape(NB * SEG)
  g_e, inv_e = _expand(src_p, et_p, dst_p, inv)

  wc1 = W1.transpose(1, 0, 2).reshape(DD, RR * DD)
  wc2 = W2.transpose(1, 0, 2).reshape(DD, RR * DD)

  y1, xr1 = _mm(x, wc1, root1, b1.reshape(1, DD))
  parts1 = _scat(y1.reshape(NB, SEG, DD), g_e, inv_e, dst5, zrows)
  h = _fuse(parts1.reshape(NB, NC, NN, DD), xr1, relu=True)

  y2, xr2 = _mm(h, wc2, root2, b2.reshape(1, DD))
  parts2 = _scat(y2.reshape(NB, SEG, DD), g_e, inv_e, dst5, zrows)
  return _fuse(parts2.reshape(NB, NC, NN, DD), xr2, relu=False)


# speedup vs baseline: 2.5061x; 1.3280x over previous
import functools

import jax
import jax.numpy as jnp
from jax import lax
from jax.experimental import pallas as pl
from jax.experimental.pallas import tpu as pltpu
from jax.experimental.pallas import tpu_sc as plsc

NN = 10000
RR = 8
DD = 128
EE = 320000
NB = 2
NC = 2
NS = 16
LL = 16
NW = NC * NS
SEG = NN * RR
EPT = EE // NW
CH = 80
NCHUNK = EPT // CH
GRP = CH // LL
RPT = NN // NS

_MESH = plsc.VectorSubcoreMesh(
    core_axis_name="c", subcore_axis_name="s", num_cores=NC, num_subcores=NS)
_SC_PARAMS = pltpu.CompilerParams(needs_layout_passes=False)


def _count_body(dst_hbm, et_hbm, out_hbm, dstb, etb, hist):
  cid = lax.axis_index("c")
  sid = lax.axis_index("s")
  wid = sid * NC + cid
  zeros = jnp.zeros((LL,), jnp.int32)
  ones = jnp.ones((LL,), jnp.int32)
  for b in range(NB):
    def zbody(i, _):
      hist[pl.ds(i * LL, LL)] = zeros
      return 0
    lax.fori_loop(0, SEG // LL, zbody, 0)
    base = b * EE + wid * EPT
    pltpu.sync_copy(dst_hbm.at[pl.ds(base, EPT)], dstb)
    pltpu.sync_copy(et_hbm.at[pl.ds(base, EPT)], etb)
    def cbody(i, _):
      sl = pl.ds(i * LL, LL)
      seg = dstb[sl] * RR + etb[sl]
      plsc.addupdate_scatter(hist, [seg], ones)
      return 0
    lax.fori_loop(0, EPT // LL, cbody, 0)
    pltpu.sync_copy(hist, out_hbm.at[pl.ds((b * NW + wid) * SEG, SEG)])


_count = pl.kernel(
    _count_body,
    out_type=jax.ShapeDtypeStruct((NB * NW * SEG,), jnp.int32),
    mesh=_MESH,
    scratch_types=[
        pltpu.VMEM((EPT,), jnp.int32),
        pltpu.VMEM((EPT,), jnp.int32),
        pltpu.VMEM((SEG,), jnp.int32),
    ],
    compiler_params=_SC_PARAMS,
)


PAD_E = 327680
EPT2 = PAD_E // NW
CH2 = 128
SUP2 = 2048
NSUP2 = EPT2 // SUP2
SUPCH2 = SUP2 // CH2
GRP2 = CH2 // LL
NBUF = 2


def _expand_body(src_hbm, et_hbm, dst_hbm, inv_hbm, g_hbm, inve_hbm,
                 srcb, etb, dstb, invt, outg, outv):
  cid = lax.axis_index("c")
  sid = lax.axis_index("s")
  wid = sid * NC + cid
  lanes = lax.iota(jnp.int32, LL)
  for b in range(NB):
    pltpu.sync_copy(inv_hbm.at[pl.ds(b * SEG, SEG)], invt)
    base = wid * EPT2
    def sup_body(s, _):
      sbase = base + s * SUP2
      pltpu.sync_copy(src_hbm.at[pl.ds(b * PAD_E + sbase, SUP2)], srcb)
      pltpu.sync_copy(et_hbm.at[pl.ds(b * PAD_E + sbase, SUP2)], etb)
      pltpu.sync_copy(dst_hbm.at[pl.ds(b * PAD_E + sbase, SUP2)], dstb)
      def ebody(i, _):
        sl = pl.ds(i * LL, LL)
        t = etb[sl]
        seg = dstb[sl] * RR + t
        iv = plsc.load_gather(invt, [seg])
        eidx = (sbase + i * LL) + lanes
        outv[sl] = jnp.where(eidx < EE, iv, 0.0)
        outg[sl] = srcb[sl] * RR + t
        return 0
      lax.fori_loop(0, SUP2 // LL, ebody, 0)
      pltpu.sync_copy(outg, g_hbm.at[pl.ds(b * PAD_E + sbase, SUP2)])
      pltpu.sync_copy(outv, inve_hbm.at[pl.ds(b * PAD_E + sbase, SUP2)])
      return 0
    lax.fori_loop(0, NSUP2, sup_body, 0)


_expand = pl.kernel(
    _expand_body,
    out_type=(jax.ShapeDtypeStruct((NB * PAD_E,), jnp.int32),
              jax.ShapeDtypeStruct((NB * PAD_E,), jnp.float32)),
    mesh=_MESH,
    scratch_types=[
        pltpu.VMEM((SUP2,), jnp.int32),
        pltpu.VMEM((SUP2,), jnp.int32),
        pltpu.VMEM((SUP2,), jnp.int32),
        pltpu.VMEM((SEG,), jnp.float32),
        pltpu.VMEM((SUP2,), jnp.int32),
        pltpu.VMEM((SUP2,), jnp.float32),
    ],
    compiler_params=_SC_PARAMS,
)


def _scat_body(y_hbm, g_hbm, inve_hbm, dst5_hbm, z_hbm,
               out_hbm, gbs, invsb, dstb, rows, acc, gsem, ssem):
  cid = lax.axis_index("c")
  sid = lax.axis_index("s")
  for b in range(NB):
    pltpu.sync_copy(z_hbm, acc.at[pl.ds(sid * RPT, RPT)])
    plsc.subcore_barrier()
    yb = y_hbm.at[b]
    base = b * PAD_E + cid * (PAD_E // NC) + sid * EPT2

    def start_g(i, rb, sem):
      pltpu.async_copy(yb.at[gbs.at[pl.ds(i * CH2, CH2)]], rb, sem)

    def step(i, rb, sem):
      pltpu.make_async_copy(yb.at[gbs.at[pl.ds(i * CH2, CH2)]], rb,
                            sem).wait()
      def sc_body(k, _):
        for j in range(LL):
          e = k * LL + j
          scal = plsc.load_gather(
              invsb, [jnp.full((LL,), i * CH2 + e, jnp.int32)])
          for q in range(DD // LL):
            sl = pl.ds(q * LL, LL)
            rb[e, sl] = rb[e, sl] * scal
        return 0
      lax.fori_loop(0, GRP2, sc_body, 0)
      pltpu.sync_copy(rb, acc.at[dstb.at[i]], add=True)

    def sup_body(s, _):
      sbase = base + s * SUP2
      pltpu.sync_copy(g_hbm.at[pl.ds(sbase, SUP2)], gbs)
      pltpu.sync_copy(inve_hbm.at[pl.ds(sbase, SUP2)], invsb)
      pltpu.sync_copy(dst5_hbm.at[b, cid, sid, pl.ds(s * SUPCH2, SUPCH2)],
                      dstb)
      r0, r1 = rows.at[0], rows.at[1]
      s0, s1 = gsem.at[0], gsem.at[1]
      start_g(0, r0, s0)
      def pbody(p, _):
        c0 = 2 * p
        start_g(c0 + 1, r1, s1)
        step(c0, r0, s0)
        @pl.when(p < SUPCH2 // 2 - 1)
        def _():
          start_g(c0 + 2, r0, s0)
        step(c0 + 1, r1, s1)
        return 0
      lax.fori_loop(0, SUPCH2 // 2, pbody, 0)
      return 0
    lax.fori_loop(0, NSUP2, sup_body, 0)
    plsc.subcore_barrier()
    pltpu.sync_copy(acc.at[pl.ds(sid * RPT, RPT)], out_hbm.at[b, cid, sid])


_scat = pl.kernel(
    _scat_body,
    out_type=jax.ShapeDtypeStruct((NB, NC, NS, RPT, DD), jnp.float32),
    mesh=_MESH,
    scratch_types=[
        pltpu.VMEM((SUP2,), jnp.int32),
        pltpu.VMEM((SUP2,), jnp.float32),
        pltpu.VMEM((SUPCH2, CH2), jnp.int32),
        pltpu.VMEM((NBUF, CH2, DD), jnp.float32),
        pltpu.VMEM_SHARED((NN, DD), jnp.float32),
        pltpu.SemaphoreType.DMA((NBUF,)),
        pltpu.SemaphoreType.DMA,
    ],
    compiler_params=_SC_PARAMS,
)


def _inv_body(cnt_ref, inv_ref):
  s = jnp.sum(cnt_ref[0], axis=0)
  inv_ref[...] = (1.0 / jnp.maximum(s, 1).astype(jnp.float32))[None]


def _inv(cnt):
  cnt = cnt.reshape(NB, NW, SEG // DD, DD)
  out = pl.pallas_call(
      _inv_body,
      out_shape=jax.ShapeDtypeStruct((NB, SEG // DD, DD), jnp.float32),
      grid=(NB,),
      in_specs=[pl.BlockSpec((1, NW, SEG // DD, DD), lambda b: (b, 0, 0, 0))],
      out_specs=pl.BlockSpec((1, SEG // DD, DD), lambda b: (b, 0, 0)),
  )(cnt)
  return out.reshape(NB, SEG)


_BM = 2000


def _mm_body(x_ref, w_ref, r_ref, bias_ref, y_ref, xr_ref):
  xb = x_ref[0]
  y_ref[...] = jnp.dot(xb, w_ref[...], preferred_element_type=jnp.float32)[None]
  xr_ref[...] = (jnp.dot(xb, r_ref[...], preferred_element_type=jnp.float32)
                 + bias_ref[...])[None]


def _mm(x, wc, root, bias):
  return pl.pallas_call(
      _mm_body,
      out_shape=(
          jax.ShapeDtypeStruct((NB, NN, RR * DD), jnp.float32),
          jax.ShapeDtypeStruct((NB, NN, DD), jnp.float32),
      ),
      grid=(NB, NN // _BM),
      in_specs=[
          pl.BlockSpec((1, _BM, DD), lambda b, i: (b, i, 0)),
          pl.BlockSpec((DD, RR * DD), lambda b, i: (0, 0)),
          pl.BlockSpec((DD, DD), lambda b, i: (0, 0)),
          pl.BlockSpec((1, DD), lambda b, i: (0, 0)),
      ],
      out_specs=(
          pl.BlockSpec((1, _BM, RR * DD), lambda b, i: (b, i, 0)),
          pl.BlockSpec((1, _BM, DD), lambda b, i: (b, i, 0)),
      ),
  )(x, wc, root, bias)


def _fuse_body(part_ref, xr_ref, o_ref, *, relu):
  s = part_ref[0, 0] + part_ref[0, 1] + xr_ref[0]
  if relu:
    s = jnp.maximum(s, 0.0)
  o_ref[...] = s[None]


def _fuse(parts, xr, relu):
  return pl.pallas_call(
      functools.partial(_fuse_body, relu=relu),
      out_shape=jax.ShapeDtypeStruct((NB, NN, DD), jnp.float32),
      grid=(NB, NN // _BM),
      in_specs=[
          pl.BlockSpec((1, NC, _BM, DD), lambda b, i: (b, 0, i, 0)),
          pl.BlockSpec((1, _BM, DD), lambda b, i: (b, i, 0)),
      ],
      out_specs=pl.BlockSpec((1, _BM, DD), lambda b, i: (b, i, 0)),
  )(parts, xr)


def kernel(x, batched_edge_index, batched_edge_type, W1, root1, b1,
           W2, root2, b2):
  src2 = batched_edge_index[:, 0, :].astype(jnp.int32)
  dst2 = batched_edge_index[:, 1, :].astype(jnp.int32)
  et2 = batched_edge_type.astype(jnp.int32)
  spread = (jnp.arange(PAD_E - EE, dtype=jnp.int32) % NN)[None, :]
  spread = jnp.broadcast_to(spread, (NB, PAD_E - EE))
  src_p = jnp.concatenate([src2, spread], axis=1).reshape(NB * PAD_E)
  dst_p = jnp.concatenate([dst2, spread], axis=1).reshape(NB * PAD_E)
  et_p = jnp.concatenate(
      [et2, jnp.zeros((NB, PAD_E - EE), jnp.int32)], axis=1
  ).reshape(NB * PAD_E)
  dst5 = dst_p.reshape(NB, NC, NS, EPT2 // CH2, CH2)
  zrows = jnp.zeros((RPT, DD), jnp.float32)

  cnt = _count(dst2.reshape(NB * EE), et2.reshape(NB * EE))
  inv = _inv(cnt).reshape(NB * SEG)
  g_e, inv_e = _expand(src_p, et_p, dst_p, inv)

  wc1 = W1.transpose(1, 0, 2).reshape(DD, RR * DD)
  wc2 = W2.transpose(1, 0, 2).reshape(DD, RR * DD)

  y1, xr1 = _mm(x, wc1, root1, b1.reshape(1, DD))
  parts1 = _scat(y1.reshape(NB, SEG, DD), g_e, inv_e, dst5, zrows)
  h = _fuse(parts1.reshape(NB, NC, NN, DD), xr1, relu=True)

  y2, xr2 = _mm(h, wc2, root2, b2.reshape(1, DD))
  parts2 = _scat(y2.reshape(NB, SEG, DD), g_e, inv_e, dst5, zrows)
  return _fuse(parts2.reshape(NB, NC, NN, DD), xr2, relu=False)


# speedup vs baseline: 2.7961x; 1.1157x over previous
import functools

import jax
import jax.numpy as jnp
from jax import lax
from jax.experimental import pallas as pl
from jax.experimental.pallas import tpu as pltpu
from jax.experimental.pallas import tpu_sc as plsc

NN = 10000
RR = 8
DD = 128
EE = 320000
NB = 2
NC = 2
NS = 16
LL = 16
NW = NC * NS
SEG = NN * RR
EPT = EE // NW
CH = 80
NCHUNK = EPT // CH
GRP = CH // LL
RPT = NN // NS

_MESH = plsc.VectorSubcoreMesh(
    core_axis_name="c", subcore_axis_name="s", num_cores=NC, num_subcores=NS)
_SC_PARAMS = pltpu.CompilerParams(needs_layout_passes=False)


def _count_body(dst_hbm, et_hbm, out_hbm, dstb, etb, hist):
  cid = lax.axis_index("c")
  sid = lax.axis_index("s")
  wid = sid * NC + cid
  zeros = jnp.zeros((LL,), jnp.int32)
  ones = jnp.ones((LL,), jnp.int32)
  for b in range(NB):
    def zbody(i, _):
      hist[pl.ds(i * LL, LL)] = zeros
      return 0
    lax.fori_loop(0, SEG // LL, zbody, 0)
    base = b * EE + wid * EPT
    pltpu.sync_copy(dst_hbm.at[pl.ds(base, EPT)], dstb)
    pltpu.sync_copy(et_hbm.at[pl.ds(base, EPT)], etb)
    def cbody(i, _):
      sl = pl.ds(i * LL, LL)
      seg = dstb[sl] * RR + etb[sl]
      plsc.addupdate_scatter(hist, [seg], ones)
      return 0
    lax.fori_loop(0, EPT // LL, cbody, 0)
    pltpu.sync_copy(hist, out_hbm.at[pl.ds((b * NW + wid) * SEG, SEG)])


_count = pl.kernel(
    _count_body,
    out_type=jax.ShapeDtypeStruct((NB * NW * SEG,), jnp.int32),
    mesh=_MESH,
    scratch_types=[
        pltpu.VMEM((EPT,), jnp.int32),
        pltpu.VMEM((EPT,), jnp.int32),
        pltpu.VMEM((SEG,), jnp.int32),
    ],
    compiler_params=_SC_PARAMS,
)


PAD_E = 322560
EPT2 = PAD_E // NW
CH2 = 96
SUP2 = 2016
NSUP2 = EPT2 // SUP2
SUPCH2 = SUP2 // CH2
GRP2 = CH2 // LL
NBUF = 3


def _expand_body(src_hbm, et_hbm, dst_hbm, inv_hbm, g_hbm, inve_hbm,
                 srcb, etb, dstb, invt, outg, outv):
  cid = lax.axis_index("c")
  sid = lax.axis_index("s")
  wid = sid * NC + cid
  lanes = lax.iota(jnp.int32, LL)
  for b in range(NB):
    pltpu.sync_copy(inv_hbm.at[pl.ds(b * SEG, SEG)], invt)
    base = wid * EPT2
    def sup_body(s, _):
      sbase = base + s * SUP2
      pltpu.sync_copy(src_hbm.at[pl.ds(b * PAD_E + sbase, SUP2)], srcb)
      pltpu.sync_copy(et_hbm.at[pl.ds(b * PAD_E + sbase, SUP2)], etb)
      pltpu.sync_copy(dst_hbm.at[pl.ds(b * PAD_E + sbase, SUP2)], dstb)
      def ebody(i, _):
        sl = pl.ds(i * LL, LL)
        t = etb[sl]
        seg = dstb[sl] * RR + t
        iv = plsc.load_gather(invt, [seg])
        eidx = (sbase + i * LL) + lanes
        outv[sl] = jnp.where(eidx < EE, iv, 0.0)
        outg[sl] = srcb[sl] * RR + t
        return 0
      lax.fori_loop(0, SUP2 // LL, ebody, 0)
      pltpu.sync_copy(outg, g_hbm.at[pl.ds(b * PAD_E + sbase, SUP2)])
      pltpu.sync_copy(outv, inve_hbm.at[pl.ds(b * PAD_E + sbase, SUP2)])
      return 0
    lax.fori_loop(0, NSUP2, sup_body, 0)


_expand = pl.kernel(
    _expand_body,
    out_type=(jax.ShapeDtypeStruct((NB * PAD_E,), jnp.int32),
              jax.ShapeDtypeStruct((NB * PAD_E,), jnp.float32)),
    mesh=_MESH,
    scratch_types=[
        pltpu.VMEM((SUP2,), jnp.int32),
        pltpu.VMEM((SUP2,), jnp.int32),
        pltpu.VMEM((SUP2,), jnp.int32),
        pltpu.VMEM((SEG,), jnp.float32),
        pltpu.VMEM((SUP2,), jnp.int32),
        pltpu.VMEM((SUP2,), jnp.float32),
    ],
    compiler_params=_SC_PARAMS,
)


def _scat_body(y_hbm, g_hbm, inve_hbm, dst6_hbm, z_hbm,
               out_hbm, gbs, invsb, dstb, rows, acc, gsem, ssem):
  cid = lax.axis_index("c")
  sid = lax.axis_index("s")
  for b in range(NB):
    pltpu.sync_copy(z_hbm, acc.at[pl.ds(sid * RPT, RPT)])
    plsc.subcore_barrier()
    yb = y_hbm.at[b]
    base = b * PAD_E + cid * (PAD_E // NC) + sid * EPT2

    def start_g(i, n):
      pltpu.async_copy(yb.at[gbs.at[pl.ds(i * CH2, CH2)]], rows.at[n],
                       gsem.at[n])

    def wait_g(i, n):
      pltpu.make_async_copy(yb.at[gbs.at[pl.ds(i * CH2, CH2)]], rows.at[n],
                            gsem.at[n]).wait()

    def start_s(i, n):
      pltpu.async_copy(rows.at[n], acc.at[dstb.at[i]], ssem.at[n], add=True)

    def wait_s(i, n):
      pltpu.make_async_copy(rows.at[n], acc.at[dstb.at[i]],
                            ssem.at[n]).wait()

    def scale(i, n):
      rb = rows.at[n]
      def sc_body(k, _):
        for j in range(LL):
          e = k * LL + j
          scal = plsc.load_gather(
              invsb, [jnp.full((LL,), i * CH2 + e, jnp.int32)])
          for q in range(DD // LL):
            sl = pl.ds(q * LL, LL)
            rb[e, sl] = rb[e, sl] * scal
        return 0
      lax.fori_loop(0, GRP2, sc_body, 0)

    def sup_body(s, _):
      sbase = base + s * SUP2
      pltpu.sync_copy(g_hbm.at[pl.ds(sbase, SUP2)], gbs)
      pltpu.sync_copy(inve_hbm.at[pl.ds(sbase, SUP2)], invsb)
      pltpu.sync_copy(dst6_hbm.at[b, cid, sid, s], dstb)
      start_g(0, 0)
      def triple(t, _):
        c = 3 * t
        @pl.when(t >= 1)
        def _():
          wait_s(c - 2, 1)
        start_g(c + 1, 1)
        wait_g(c, 0)
        scale(c, 0)
        start_s(c, 0)
        @pl.when(t >= 1)
        def _():
          wait_s(c - 1, 2)
        start_g(c + 2, 2)
        wait_g(c + 1, 1)
        scale(c + 1, 1)
        start_s(c + 1, 1)
        wait_s(c, 0)
        @pl.when(t < SUPCH2 // 3 - 1)
        def _():
          start_g(c + 3, 0)
        wait_g(c + 2, 2)
        scale(c + 2, 2)
        start_s(c + 2, 2)
        return 0
      lax.fori_loop(0, SUPCH2 // 3, triple, 0)
      wait_s(SUPCH2 - 2, 1)
      wait_s(SUPCH2 - 1, 2)
      return 0
    lax.fori_loop(0, NSUP2, sup_body, 0)
    plsc.subcore_barrier()
    pltpu.sync_copy(acc.at[pl.ds(sid * RPT, RPT)], out_hbm.at[b, cid, sid])


_scat = pl.kernel(
    _scat_body,
    out_type=jax.ShapeDtypeStruct((NB, NC, NS, RPT, DD), jnp.float32),
    mesh=_MESH,
    scratch_types=[
        pltpu.VMEM((SUP2,), jnp.int32),
        pltpu.VMEM((SUP2,), jnp.float32),
        pltpu.VMEM((SUPCH2, CH2), jnp.int32),
        pltpu.VMEM((NBUF, CH2, DD), jnp.float32),
        pltpu.VMEM_SHARED((NN, DD), jnp.float32),
        pltpu.SemaphoreType.DMA((NBUF,)),
        pltpu.SemaphoreType.DMA((NBUF,)),
    ],
    compiler_params=_SC_PARAMS,
)


def _inv_body(cnt_ref, inv_ref):
  s = jnp.sum(cnt_ref[0], axis=0)
  inv_ref[...] = (1.0 / jnp.maximum(s, 1).astype(jnp.float32))[None]


def _inv(cnt):
  cnt = cnt.reshape(NB, NW, SEG // DD, DD)
  out = pl.pallas_call(
      _inv_body,
      out_shape=jax.ShapeDtypeStruct((NB, SEG // DD, DD), jnp.float32),
      grid=(NB,),
      in_specs=[pl.BlockSpec((1, NW, SEG // DD, DD), lambda b: (b, 0, 0, 0))],
      out_specs=pl.BlockSpec((1, SEG // DD, DD), lambda b: (b, 0, 0)),
  )(cnt)
  return out.reshape(NB, SEG)


_BM = 2000


def _mm_body(x_ref, w_ref, r_ref, bias_ref, y_ref, xr_ref):
  xb = x_ref[0]
  y_ref[...] = jnp.dot(xb, w_ref[...], preferred_element_type=jnp.float32)[None]
  xr_ref[...] = (jnp.dot(xb, r_ref[...], preferred_element_type=jnp.float32)
                 + bias_ref[...])[None]


def _mm(x, wc, root, bias):
  return pl.pallas_call(
      _mm_body,
      out_shape=(
          jax.ShapeDtypeStruct((NB, NN, RR * DD), jnp.float32),
          jax.ShapeDtypeStruct((NB, NN, DD), jnp.float32),
      ),
      grid=(NB, NN // _BM),
      in_specs=[
          pl.BlockSpec((1, _BM, DD), lambda b, i: (b, i, 0)),
          pl.BlockSpec((DD, RR * DD), lambda b, i: (0, 0)),
          pl.BlockSpec((DD, DD), lambda b, i: (0, 0)),
          pl.BlockSpec((1, DD), lambda b, i: (0, 0)),
      ],
      out_specs=(
          pl.BlockSpec((1, _BM, RR * DD), lambda b, i: (b, i, 0)),
          pl.BlockSpec((1, _BM, DD), lambda b, i: (b, i, 0)),
      ),
  )(x, wc, root, bias)


def _fuse_body(part_ref, xr_ref, o_ref, *, relu):
  s = part_ref[0, 0] + part_ref[0, 1] + xr_ref[0]
  if relu:
    s = jnp.maximum(s, 0.0)
  o_ref[...] = s[None]


def _fuse(parts, xr, relu):
  return pl.pallas_call(
      functools.partial(_fuse_body, relu=relu),
      out_shape=jax.ShapeDtypeStruct((NB, NN, DD), jnp.float32),
      grid=(NB, NN // _BM),
      in_specs=[
          pl.BlockSpec((1, NC, _BM, DD), lambda b, i: (b, 0, i, 0)),
          pl.BlockSpec((1, _BM, DD), lambda b, i: (b, i, 0)),
      ],
      out_specs=pl.BlockSpec((1, _BM, DD), lambda b, i: (b, i, 0)),
  )(parts, xr)


def kernel(x, batched_edge_index, batched_edge_type, W1, root1, b1,
           W2, root2, b2):
  src2 = batched_edge_index[:, 0, :].astype(jnp.int32)
  dst2 = batched_edge_index[:, 1, :].astype(jnp.int32)
  et2 = batched_edge_type.astype(jnp.int32)
  spread = (jnp.arange(PAD_E - EE, dtype=jnp.int32) % NN)[None, :]
  spread = jnp.broadcast_to(spread, (NB, PAD_E - EE))
  src_p = jnp.concatenate([src2, spread], axis=1).reshape(NB * PAD_E)
  dst_p = jnp.concatenate([dst2, spread], axis=1).reshape(NB * PAD_E)
  et_p = jnp.concatenate(
      [et2, jnp.zeros((NB, PAD_E - EE), jnp.int32)], axis=1
  ).reshape(NB * PAD_E)
  dst6 = dst_p.reshape(NB, NC, NS, NSUP2, SUPCH2, CH2)
  zrows = jnp.zeros((RPT, DD), jnp.float32)

  cnt = _count(dst2.reshape(NB * EE), et2.reshape(NB * EE))
  inv = _inv(cnt).reshape(NB * SEG)
  g_e, inv_e = _expand(src_p, et_p, dst_p, inv)

  wc1 = W1.transpose(1, 0, 2).reshape(DD, RR * DD)
  wc2 = W2.transpose(1, 0, 2).reshape(DD, RR * DD)

  y1, xr1 = _mm(x, wc1, root1, b1.reshape(1, DD))
  parts1 = _scat(y1.reshape(NB, SEG, DD), g_e, inv_e, dst6, zrows)
  h = _fuse(parts1.reshape(NB, NC, NN, DD), xr1, relu=True)

  y2, xr2 = _mm(h, wc2, root2, b2.reshape(1, DD))
  parts2 = _scat(y2.reshape(NB, SEG, DD), g_e, inv_e, dst6, zrows)
  return _fuse(parts2.reshape(NB, NC, NN, DD), xr2, relu=False)


# speedup vs baseline: 2.8831x; 1.0311x over previous
import functools

import jax
import jax.numpy as jnp
from jax import lax
from jax.experimental import pallas as pl
from jax.experimental.pallas import tpu as pltpu
from jax.experimental.pallas import tpu_sc as plsc

NN = 10000
RR = 8
DD = 128
EE = 320000
NB = 2
NC = 2
NS = 16
LL = 16
NW = NC * NS
SEG = NN * RR
EPT = EE // NW
CH = 80
NCHUNK = EPT // CH
GRP = CH // LL
RPT = NN // NS

_MESH = plsc.VectorSubcoreMesh(
    core_axis_name="c", subcore_axis_name="s", num_cores=NC, num_subcores=NS)
_SC_PARAMS = pltpu.CompilerParams(needs_layout_passes=False)


def _count_body(dst_hbm, et_hbm, zseg_hbm, out_hbm, dstb, etb, hist):
  cid = lax.axis_index("c")
  sid = lax.axis_index("s")
  wid = sid * NC + cid
  ones = jnp.ones((LL,), jnp.int32)
  for b in range(NB):
    pltpu.sync_copy(zseg_hbm, hist)
    base = b * EE + wid * EPT
    pltpu.sync_copy(dst_hbm.at[pl.ds(base, EPT)], dstb)
    pltpu.sync_copy(et_hbm.at[pl.ds(base, EPT)], etb)
    def cbody(i, _):
      sl = pl.ds(i * LL, LL)
      seg = dstb[sl] * RR + etb[sl]
      plsc.addupdate_scatter(hist, [seg], ones)
      return 0
    lax.fori_loop(0, EPT // LL, cbody, 0)
    pltpu.sync_copy(hist, out_hbm.at[pl.ds((b * NW + wid) * SEG, SEG)])


_count = pl.kernel(
    _count_body,
    out_type=jax.ShapeDtypeStruct((NB * NW * SEG,), jnp.int32),
    mesh=_MESH,
    scratch_types=[
        pltpu.VMEM((EPT,), jnp.int32),
        pltpu.VMEM((EPT,), jnp.int32),
        pltpu.VMEM((SEG,), jnp.int32),
    ],
    compiler_params=_SC_PARAMS,
)


PAD_E = 322560
EPT2 = PAD_E // NW
CH2 = 96
SUP2 = 2016
NSUP2 = EPT2 // SUP2
SUPCH2 = SUP2 // CH2
GRP2 = CH2 // LL
NBUF = 3


def _expand_body(src_hbm, et_hbm, dst_hbm, inv_hbm, g_hbm, inve_hbm,
                 srcb, etb, dstb, invt, outg, outv):
  cid = lax.axis_index("c")
  sid = lax.axis_index("s")
  wid = sid * NC + cid
  lanes = lax.iota(jnp.int32, LL)
  for b in range(NB):
    pltpu.sync_copy(inv_hbm.at[pl.ds(b * SEG, SEG)], invt)
    base = wid * EPT2
    def sup_body(s, _):
      sbase = base + s * SUP2
      pltpu.sync_copy(src_hbm.at[pl.ds(b * PAD_E + sbase, SUP2)], srcb)
      pltpu.sync_copy(et_hbm.at[pl.ds(b * PAD_E + sbase, SUP2)], etb)
      pltpu.sync_copy(dst_hbm.at[pl.ds(b * PAD_E + sbase, SUP2)], dstb)
      def ebody(i, _):
        sl = pl.ds(i * LL, LL)
        t = etb[sl]
        seg = dstb[sl] * RR + t
        iv = plsc.load_gather(invt, [seg])
        eidx = (sbase + i * LL) + lanes
        outv[sl] = jnp.where(eidx < EE, iv, 0.0)
        outg[sl] = srcb[sl] * RR + t
        return 0
      lax.fori_loop(0, SUP2 // LL, ebody, 0)
      pltpu.sync_copy(outg, g_hbm.at[pl.ds(b * PAD_E + sbase, SUP2)])
      pltpu.sync_copy(outv, inve_hbm.at[pl.ds(b * PAD_E + sbase, SUP2)])
      return 0
    lax.fori_loop(0, NSUP2, sup_body, 0)


_expand = pl.kernel(
    _expand_body,
    out_type=(jax.ShapeDtypeStruct((NB * PAD_E,), jnp.int32),
              jax.ShapeDtypeStruct((NB * PAD_E,), jnp.float32)),
    mesh=_MESH,
    scratch_types=[
        pltpu.VMEM((SUP2,), jnp.int32),
        pltpu.VMEM((SUP2,), jnp.int32),
        pltpu.VMEM((SUP2,), jnp.int32),
        pltpu.VMEM((SEG,), jnp.float32),
        pltpu.VMEM((SUP2,), jnp.int32),
        pltpu.VMEM((SUP2,), jnp.float32),
    ],
    compiler_params=_SC_PARAMS,
)


def _scat_body(y_hbm, g_hbm, inve_hbm, dst6_hbm, z_hbm,
               out_hbm, gbs, invsb, dstb, rows, acc, gsem, ssem):
  cid = lax.axis_index("c")
  sid = lax.axis_index("s")
  for b in range(NB):
    pltpu.sync_copy(z_hbm, acc.at[pl.ds(sid * RPT, RPT)])
    plsc.subcore_barrier()
    yb = y_hbm.at[b]
    base = b * PAD_E + cid * (PAD_E // NC) + sid * EPT2

    def start_g(i, n):
      pltpu.async_copy(yb.at[gbs.at[pl.ds(i * CH2, CH2)]], rows.at[n],
                       gsem.at[n])

    def wait_g(i, n):
      pltpu.make_async_copy(yb.at[gbs.at[pl.ds(i * CH2, CH2)]], rows.at[n],
                            gsem.at[n]).wait()

    def start_s(i, n):
      pltpu.async_copy(rows.at[n], acc.at[dstb.at[i]], ssem.at[n], add=True)

    def wait_s(i, n):
      pltpu.make_async_copy(rows.at[n], acc.at[dstb.at[i]],
                            ssem.at[n]).wait()

    def scale(i, n):
      rb = rows.at[n]
      def sc_body(k, _):
        for j in range(LL):
          e = k * LL + j
          scal = plsc.load_gather(
              invsb, [jnp.full((LL,), i * CH2 + e, jnp.int32)])
          for q in range(DD // LL):
            sl = pl.ds(q * LL, LL)
            rb[e, sl] = rb[e, sl] * scal
        return 0
      lax.fori_loop(0, GRP2, sc_body, 0)

    def sup_body(s, _):
      sbase = base + s * SUP2
      pltpu.sync_copy(g_hbm.at[pl.ds(sbase, SUP2)], gbs)
      pltpu.sync_copy(inve_hbm.at[pl.ds(sbase, SUP2)], invsb)
      pltpu.sync_copy(dst6_hbm.at[b, cid, sid, s], dstb)
      start_g(0, 0)
      def triple(t, _):
        c = 3 * t
        @pl.when(t >= 1)
        def _():
          wait_s(c - 2, 1)
        start_g(c + 1, 1)
        wait_g(c, 0)
        scale(c, 0)
        start_s(c, 0)
        @pl.when(t >= 1)
        def _():
          wait_s(c - 1, 2)
        start_g(c + 2, 2)
        wait_g(c + 1, 1)
        scale(c + 1, 1)
        start_s(c + 1, 1)
        wait_s(c, 0)
        @pl.when(t < SUPCH2 // 3 - 1)
        def _():
          start_g(c + 3, 0)
        wait_g(c + 2, 2)
        scale(c + 2, 2)
        start_s(c + 2, 2)
        return 0
      lax.fori_loop(0, SUPCH2 // 3, triple, 0)
      wait_s(SUPCH2 - 2, 1)
      wait_s(SUPCH2 - 1, 2)
      return 0
    lax.fori_loop(0, NSUP2, sup_body, 0)
    plsc.subcore_barrier()
    pltpu.sync_copy(acc.at[pl.ds(sid * RPT, RPT)], out_hbm.at[b, cid, sid])


_scat = pl.kernel(
    _scat_body,
    out_type=jax.ShapeDtypeStruct((NB, NC, NS, RPT, DD), jnp.float32),
    mesh=_MESH,
    scratch_types=[
        pltpu.VMEM((SUP2,), jnp.int32),
        pltpu.VMEM((SUP2,), jnp.float32),
        pltpu.VMEM((SUPCH2, CH2), jnp.int32),
        pltpu.VMEM((NBUF, CH2, DD), jnp.float32),
        pltpu.VMEM_SHARED((NN, DD), jnp.float32),
        pltpu.SemaphoreType.DMA((NBUF,)),
        pltpu.SemaphoreType.DMA((NBUF,)),
    ],
    compiler_params=_SC_PARAMS,
)


def _inv_body(cnt_ref, inv_ref):
  s = jnp.sum(cnt_ref[0], axis=0)
  inv_ref[...] = (1.0 / jnp.maximum(s, 1).astype(jnp.float32))[None]


def _inv(cnt):
  cnt = cnt.reshape(NB, NW, SEG // DD, DD)
  out = pl.pallas_call(
      _inv_body,
      out_shape=jax.ShapeDtypeStruct((NB, SEG // DD, DD), jnp.float32),
      grid=(NB,),
      in_specs=[pl.BlockSpec((1, NW, SEG // DD, DD), lambda b: (b, 0, 0, 0))],
      out_specs=pl.BlockSpec((1, SEG // DD, DD), lambda b: (b, 0, 0)),
  )(cnt)
  return out.reshape(NB, SEG)


_BM = 2000


def _mm_body(x_ref, w_ref, r_ref, bias_ref, y_ref, xr_ref):
  xb = x_ref[0]
  y_ref[...] = jnp.dot(xb, w_ref[...], preferred_element_type=jnp.float32)[None]
  xr_ref[...] = (jnp.dot(xb, r_ref[...], preferred_element_type=jnp.float32)
                 + bias_ref[...])[None]


def _mm(x, wc, root, bias):
  return pl.pallas_call(
      _mm_body,
      out_shape=(
          jax.ShapeDtypeStruct((NB, NN, RR * DD), jnp.float32),
          jax.ShapeDtypeStruct((NB, NN, DD), jnp.float32),
      ),
      grid=(NB, NN // _BM),
      in_specs=[
          pl.BlockSpec((1, _BM, DD), lambda b, i: (b, i, 0)),
          pl.BlockSpec((DD, RR * DD), lambda b, i: (0, 0)),
          pl.BlockSpec((DD, DD), lambda b, i: (0, 0)),
          pl.BlockSpec((1, DD), lambda b, i: (0, 0)),
      ],
      out_specs=(
          pl.BlockSpec((1, _BM, RR * DD), lambda b, i: (b, i, 0)),
          pl.BlockSpec((1, _BM, DD), lambda b, i: (b, i, 0)),
      ),
  )(x, wc, root, bias)


def _mid_body(part_ref, xr_ref, w_ref, r_ref, bias_ref, y_ref, xr2_ref):
  h = jnp.maximum(part_ref[0, 0] + part_ref[0, 1] + xr_ref[0], 0.0)
  y_ref[...] = jnp.dot(h, w_ref[...], preferred_element_type=jnp.float32)[None]
  xr2_ref[...] = (jnp.dot(h, r_ref[...], preferred_element_type=jnp.float32)
                  + bias_ref[...])[None]


def _mid(parts, xr, wc, root, bias):
  return pl.pallas_call(
      _mid_body,
      out_shape=(
          jax.ShapeDtypeStruct((NB, NN, RR * DD), jnp.float32),
          jax.ShapeDtypeStruct((NB, NN, DD), jnp.float32),
      ),
      grid=(NB, NN // _BM),
      in_specs=[
          pl.BlockSpec((1, NC, _BM, DD), lambda b, i: (b, 0, i, 0)),
          pl.BlockSpec((1, _BM, DD), lambda b, i: (b, i, 0)),
          pl.BlockSpec((DD, RR * DD), lambda b, i: (0, 0)),
          pl.BlockSpec((DD, DD), lambda b, i: (0, 0)),
          pl.BlockSpec((1, DD), lambda b, i: (0, 0)),
      ],
      out_specs=(
          pl.BlockSpec((1, _BM, RR * DD), lambda b, i: (b, i, 0)),
          pl.BlockSpec((1, _BM, DD), lambda b, i: (b, i, 0)),
      ),
  )(parts, xr, wc, root, bias)


def _fuse_body(part_ref, xr_ref, o_ref, *, relu):
  s = part_ref[0, 0] + part_ref[0, 1] + xr_ref[0]
  if relu:
    s = jnp.maximum(s, 0.0)
  o_ref[...] = s[None]


def _fuse(parts, xr, relu):
  return pl.pallas_call(
      functools.partial(_fuse_body, relu=relu),
      out_shape=jax.ShapeDtypeStruct((NB, NN, DD), jnp.float32),
      grid=(NB, NN // _BM),
      in_specs=[
          pl.BlockSpec((1, NC, _BM, DD), lambda b, i: (b, 0, i, 0)),
          pl.BlockSpec((1, _BM, DD), lambda b, i: (b, i, 0)),
      ],
      out_specs=pl.BlockSpec((1, _BM, DD), lambda b, i: (b, i, 0)),
  )(parts, xr)


def kernel(x, batched_edge_index, batched_edge_type, W1, root1, b1,
           W2, root2, b2):
  src2 = batched_edge_index[:, 0, :].astype(jnp.int32)
  dst2 = batched_edge_index[:, 1, :].astype(jnp.int32)
  et2 = batched_edge_type.astype(jnp.int32)
  spread = (jnp.arange(PAD_E - EE, dtype=jnp.int32) % NN)[None, :]
  spread = jnp.broadcast_to(spread, (NB, PAD_E - EE))
  src_p = jnp.concatenate([src2, spread], axis=1).reshape(NB * PAD_E)
  dst_p = jnp.concatenate([dst2, spread], axis=1).reshape(NB * PAD_E)
  et_p = jnp.concatenate(
      [et2, jnp.zeros((NB, PAD_E - EE), jnp.int32)], axis=1
  ).reshape(NB * PAD_E)
  dst6 = dst_p.reshape(NB, NC, NS, NSUP2, SUPCH2, CH2)
  zrows = jnp.zeros((RPT, DD), jnp.float32)
  zseg = jnp.zeros((SEG,), jnp.int32)

  cnt = _count(dst2.reshape(NB * EE), et2.reshape(NB * EE), zseg)
  inv = _inv(cnt).reshape(NB * SEG)
  g_e, inv_e = _expand(src_p, et_p, dst_p, inv)

  wc1 = W1.transpose(1, 0, 2).reshape(DD, RR * DD)
  wc2 = W2.transpose(1, 0, 2).reshape(DD, RR * DD)

  y1, xr1 = _mm(x, wc1, root1, b1.reshape(1, DD))
  parts1 = _scat(y1.reshape(NB, SEG, DD), g_e, inv_e, dst6, zrows)
  y2, xr2 = _mid(parts1.reshape(NB, NC, NN, DD), xr1, wc2, root2,
                 b2.reshape(1, DD))
  parts2 = _scat(y2.reshape(NB, SEG, DD), g_e, inv_e, dst6, zrows)
  return _fuse(parts2.reshape(NB, NC, NN, DD), xr2, relu=False)
